# Initial kernel scaffold; baseline (speedup 1.0000x reference)
import functools

import jax
import jax.numpy as jnp
from jax import lax
from jax.experimental import pallas as pl
from jax.experimental.pallas import tpu as pltpu
from jax.experimental.pallas import tpu_sc as plsc

N_NODES = 10000
N_EDGES = 320000
D = 128

NW = 32
EPW = N_EDGES // NW
C = 80
NCHUNK = EPW // C
G = C // 16


def _body(x_hbm, src_hbm, dst_hbm, out_hbm, idx_s, idx_d, u, v, o, sem):
    wid = lax.axis_index("s") * 2 + lax.axis_index("c")
    lanes = lax.iota(jnp.int32, 16)

    def chunk_body(c, _):
        base = wid * EPW + c * C
        pltpu.sync_copy(src_hbm.at[pl.ds(base, C)], idx_s)
        pltpu.sync_copy(dst_hbm.at[pl.ds(base, C)], idx_d)
        pltpu.async_copy(x_hbm.at[idx_s], u, sem).wait()
        pltpu.async_copy(x_hbm.at[idx_d], v, sem).wait()

        def group_body(g, _):
            ev = g * 16 + lanes
            acc = jnp.zeros((16,), jnp.float32)
            for d in range(D):
                dv = jnp.full((16,), d, jnp.int32)
                ud = plsc.load_gather(u, [ev, dv])
                vd = plsc.load_gather(v, [ev, dv])
                acc = acc + ud * vd
            o[pl.ds(g * 16, 16)] = acc
            return 0

        lax.fori_loop(0, G, group_body, 0)
        pltpu.sync_copy(o, out_hbm.at[pl.ds(base, C)])
        return 0

    lax.fori_loop(0, NCHUNK, chunk_body, 0)


@jax.jit
def _run(x, src, dst):
    mesh = plsc.VectorSubcoreMesh(core_axis_name="c", subcore_axis_name="s")
    k = functools.partial(
        pl.kernel,
        mesh=mesh,
        out_type=jax.ShapeDtypeStruct((N_EDGES,), jnp.float32),
        scratch_types=[
            pltpu.VMEM((C,), jnp.int32),
            pltpu.VMEM((C,), jnp.int32),
            pltpu.VMEM((C, D), jnp.float32),
            pltpu.VMEM((C, D), jnp.float32),
            pltpu.VMEM((C,), jnp.float32),
            pltpu.SemaphoreType.DMA,
        ],
    )(_body)
    return k(x, src, dst)


def kernel(x, edge_index):
    src = edge_index[0].astype(jnp.int32)
    dst = edge_index[1].astype(jnp.int32)
    out = _run(x, src, dst)
    return out.reshape(N_EDGES, 1)


# speedup vs baseline: 2.9069x; 2.9069x over previous
import functools

import jax
import jax.numpy as jnp
from jax import lax
from jax.experimental import pallas as pl
from jax.experimental.pallas import tpu as pltpu
from jax.experimental.pallas import tpu_sc as plsc

N_NODES = 10000
N_EDGES = 320000
D = 128

NW = 32
EPW = N_EDGES // NW
C = 80
NCHUNK = EPW // C
G = C // 16


def _body(x_hbm, src_hbm, dst_hbm, out_hbm, idx_s, idx_d, u, v, o, p, sem):
    wid = lax.axis_index("s") * 2 + lax.axis_index("c")
    lanes = lax.iota(jnp.int32, 16)

    def chunk_body(c, _):
        base = wid * EPW + c * C
        pltpu.sync_copy(src_hbm.at[pl.ds(base, C)], idx_s)
        pltpu.sync_copy(dst_hbm.at[pl.ds(base, C)], idx_d)
        pltpu.async_copy(x_hbm.at[idx_s], u, sem).wait()
        pltpu.async_copy(x_hbm.at[idx_d], v, sem).wait()

        def group_body(g, _):
            for e16 in range(16):
                e = g * 16 + e16
                acc = u[e, pl.ds(0, 16)] * v[e, pl.ds(0, 16)]
                for k in range(1, D // 16):
                    acc = acc + u[e, pl.ds(k * 16, 16)] * v[e, pl.ds(k * 16, 16)]
                p[pl.ds(e16 * 16, 16)] = acc
            res = jnp.zeros((16,), jnp.float32)
            for l in range(16):
                res = res + plsc.load_gather(p, [lanes * 16 + l])
            o[pl.ds(g * 16, 16)] = res
            return 0

        lax.fori_loop(0, G, group_body, 0)
        pltpu.sync_copy(o, out_hbm.at[pl.ds(base, C)])
        return 0

    lax.fori_loop(0, NCHUNK, chunk_body, 0)


@jax.jit
def _run(x, src, dst):
    mesh = plsc.VectorSubcoreMesh(core_axis_name="c", subcore_axis_name="s")
    k = functools.partial(
        pl.kernel,
        mesh=mesh,
        compiler_params=pltpu.CompilerParams(needs_layout_passes=False),
        out_type=jax.ShapeDtypeStruct((N_EDGES,), jnp.float32),
        scratch_types=[
            pltpu.VMEM((C,), jnp.int32),
            pltpu.VMEM((C,), jnp.int32),
            pltpu.VMEM((C, D), jnp.float32),
            pltpu.VMEM((C, D), jnp.float32),
            pltpu.VMEM((C,), jnp.float32),
            pltpu.VMEM((256,), jnp.float32),
            pltpu.SemaphoreType.DMA,
        ],
    )(_body)
    return k(x, src, dst)


def kernel(x, edge_index):
    src = edge_index[0].astype(jnp.int32)
    dst = edge_index[1].astype(jnp.int32)
    out = _run(x, src, dst)
    return out.reshape(N_EDGES, 1)


# speedup vs baseline: 7.2077x; 2.4795x over previous
import functools

import jax
import jax.numpy as jnp
from jax import lax
from jax.experimental import pallas as pl
from jax.experimental.pallas import tpu as pltpu
from jax.experimental.pallas import tpu_sc as plsc

N_NODES = 10000
N_EDGES = 320000
D = 128

NW = 32
EPW = N_EDGES // NW
C = 80
NCHUNK = EPW // C
G = C // 16


def _body(x_hbm, src_hbm, dst_hbm, out_hbm, idx_s, idx_d, u, v, o, p,
          sem_g, sem_i, sem_o):
    wid = lax.axis_index("s") * 2 + lax.axis_index("c")
    lanes = lax.iota(jnp.int32, 16)
    w0 = wid * EPW

    def issue_gather(b):
        pltpu.async_copy(x_hbm.at[idx_s.at[b]], u.at[b], sem_g)
        pltpu.async_copy(x_hbm.at[idx_d.at[b]], v.at[b], sem_g)

    def wait_gather():
        pltpu.make_async_copy(x_hbm.at[pl.ds(0, C)], u.at[0], sem_g).wait()
        pltpu.make_async_copy(x_hbm.at[pl.ds(0, C)], v.at[0], sem_g).wait()

    def issue_idx(c, b):
        base = w0 + c * C
        pltpu.async_copy(src_hbm.at[pl.ds(base, C)], idx_s.at[b], sem_i)
        pltpu.async_copy(dst_hbm.at[pl.ds(base, C)], idx_d.at[b], sem_i)

    def wait_idx():
        pltpu.make_async_copy(src_hbm.at[pl.ds(0, C)], idx_s.at[0], sem_i).wait()
        pltpu.make_async_copy(dst_hbm.at[pl.ds(0, C)], idx_d.at[0], sem_i).wait()

    def wait_out():
        pltpu.make_async_copy(out_hbm.at[pl.ds(0, C)], o.at[0], sem_o).wait()

    pltpu.sync_copy(src_hbm.at[pl.ds(w0, C)], idx_s.at[0])
    pltpu.sync_copy(dst_hbm.at[pl.ds(w0, C)], idx_d.at[0])
    issue_gather(0)
    issue_idx(1, 1)

    def chunk_body(c, _):
        b = lax.rem(c, 2)
        nb = 1 - b
        wait_gather()

        @pl.when(c + 1 < NCHUNK)
        def _():
            wait_idx()
            issue_gather(nb)

        @pl.when(c + 2 < NCHUNK)
        def _():
            issue_idx(c + 2, b)

        @pl.when(c >= 2)
        def _():
            wait_out()

        ub, vb, ob = u.at[b], v.at[b], o.at[b]

        def group_body(g, _):
            for e16 in range(16):
                e = g * 16 + e16
                acc = ub[e, pl.ds(0, 16)] * vb[e, pl.ds(0, 16)]
                for k in range(1, D // 16):
                    acc = acc + ub[e, pl.ds(k * 16, 16)] * vb[e, pl.ds(k * 16, 16)]
                p[pl.ds(e16 * 16, 16)] = acc
            res = jnp.zeros((16,), jnp.float32)
            for l in range(16):
                res = res + plsc.load_gather(p, [lanes * 16 + l])
            ob[pl.ds(g * 16, 16)] = res
            return 0

        lax.fori_loop(0, G, group_body, 0)
        pltpu.async_copy(ob, out_hbm.at[pl.ds(w0 + c * C, C)], sem_o)
        return 0

    lax.fori_loop(0, NCHUNK, chunk_body, 0)
    wait_out()
    wait_out()


@jax.jit
def _run(x, src, dst):
    mesh = plsc.VectorSubcoreMesh(core_axis_name="c", subcore_axis_name="s")
    k = functools.partial(
        pl.kernel,
        mesh=mesh,
        compiler_params=pltpu.CompilerParams(needs_layout_passes=False),
        out_type=jax.ShapeDtypeStruct((N_EDGES,), jnp.float32),
        scratch_types=[
            pltpu.VMEM((2, C), jnp.int32),
            pltpu.VMEM((2, C), jnp.int32),
            pltpu.VMEM((2, C, D), jnp.float32),
            pltpu.VMEM((2, C, D), jnp.float32),
            pltpu.VMEM((2, C), jnp.float32),
            pltpu.VMEM((256,), jnp.float32),
            pltpu.SemaphoreType.DMA,
            pltpu.SemaphoreType.DMA,
            pltpu.SemaphoreType.DMA,
        ],
    )(_body)
    return k(x, src, dst)


def kernel(x, edge_index):
    src = edge_index[0].astype(jnp.int32)
    dst = edge_index[1].astype(jnp.int32)
    out = _run(x, src, dst)
    return out.reshape(N_EDGES, 1)


# speedup vs baseline: 7.4159x; 1.0289x over previous
import functools

import jax
import jax.numpy as jnp
from jax import lax
from jax.experimental import pallas as pl
from jax.experimental.pallas import tpu as pltpu
from jax.experimental.pallas import tpu_sc as plsc

N_NODES = 10000
N_EDGES = 320000
D = 128

NW = 32
EPW = N_EDGES // NW
C = 80
NCHUNK = EPW // C
G = C // 16


def _body(x_hbm, src_hbm, dst_hbm, out_hbm, idx_s, idx_d, u, v, o, p,
          sem_g, sem_i, sem_o):
    wid = lax.axis_index("s") * 2 + lax.axis_index("c")
    lanes = lax.iota(jnp.int32, 16)
    w0 = wid * EPW

    def issue_gather(b):
        pltpu.async_copy(x_hbm.at[idx_s.at[b]], u.at[b], sem_g)
        pltpu.async_copy(x_hbm.at[idx_d.at[b]], v.at[b], sem_g)

    def wait_gather():
        pltpu.make_async_copy(x_hbm.at[pl.ds(0, C)], u.at[0], sem_g).wait()
        pltpu.make_async_copy(x_hbm.at[pl.ds(0, C)], v.at[0], sem_g).wait()

    def issue_idx(c, b):
        base = w0 + c * C
        pltpu.async_copy(src_hbm.at[pl.ds(base, C)], idx_s.at[b], sem_i)
        pltpu.async_copy(dst_hbm.at[pl.ds(base, C)], idx_d.at[b], sem_i)

    def wait_idx():
        pltpu.make_async_copy(src_hbm.at[pl.ds(0, C)], idx_s.at[0], sem_i).wait()
        pltpu.make_async_copy(dst_hbm.at[pl.ds(0, C)], idx_d.at[0], sem_i).wait()

    def wait_out():
        pltpu.make_async_copy(out_hbm.at[pl.ds(0, C)], o.at[0], sem_o).wait()

    pltpu.sync_copy(src_hbm.at[pl.ds(w0, C)], idx_s.at[0])
    pltpu.sync_copy(dst_hbm.at[pl.ds(w0, C)], idx_d.at[0])
    issue_gather(0)
    issue_idx(1, 1)

    def chunk_body(c, _):
        b = lax.rem(c, 2)
        nb = 1 - b
        wait_gather()

        @pl.when(c + 1 < NCHUNK)
        def _():
            wait_idx()
            issue_gather(nb)

        @pl.when(c + 2 < NCHUNK)
        def _():
            issue_idx(c + 2, b)

        @pl.when(c >= 2)
        def _():
            wait_out()

        ub, vb, ob = u.at[b], v.at[b], o.at[b]

        def group_body(g, _):
            for e16 in range(16):
                e = g * 16 + e16
                pr = []
                for k in range(D // 32):
                    uk = plsc.bitcast(ub[e, pl.ds(k * 16, 16)], jnp.bfloat16)
                    vk = plsc.bitcast(vb[e, pl.ds(k * 16, 16)], jnp.bfloat16)
                    pr.append(uk * vk)
                s = (pr[0] + pr[1]) + (pr[2] + pr[3])
                a0, a1 = plsc.unpack(s, format=plsc.PackFormat.INTERLEAVED)
                p[pl.ds(e16 * 16, 16)] = a0 + a1
            res = jnp.zeros((16,), jnp.float32)
            for l in range(16):
                res = res + plsc.load_gather(p, [lanes * 16 + l])
            ob[pl.ds(g * 16, 16)] = res
            return 0

        lax.fori_loop(0, G, group_body, 0)
        pltpu.async_copy(ob, out_hbm.at[pl.ds(w0 + c * C, C)], sem_o)
        return 0

    lax.fori_loop(0, NCHUNK, chunk_body, 0)
    wait_out()
    wait_out()


@jax.jit
def _run(x, src, dst):
    mesh = plsc.VectorSubcoreMesh(core_axis_name="c", subcore_axis_name="s")
    k = functools.partial(
        pl.kernel,
        mesh=mesh,
        compiler_params=pltpu.CompilerParams(
            needs_layout_passes=False, use_tc_tiling_on_sc=False),
        out_type=jax.ShapeDtypeStruct((N_EDGES,), jnp.float32),
        scratch_types=[
            pltpu.VMEM((2, C), jnp.int32),
            pltpu.VMEM((2, C), jnp.int32),
            pltpu.VMEM((2, C, D // 2), jnp.int32),
            pltpu.VMEM((2, C, D // 2), jnp.int32),
            pltpu.VMEM((2, C), jnp.float32),
            pltpu.VMEM((256,), jnp.float32),
            pltpu.SemaphoreType.DMA,
            pltpu.SemaphoreType.DMA,
            pltpu.SemaphoreType.DMA,
        ],
    )(_body)
    return k(x, src, dst)


def kernel(x, edge_index):
    src = edge_index[0].astype(jnp.int32)
    dst = edge_index[1].astype(jnp.int32)
    xb = x.astype(jnp.bfloat16)
    xi = lax.bitcast_convert_type(xb.reshape(N_NODES, D // 2, 2), jnp.int32)
    out = _run(xi, src, dst)
    return out.reshape(N_EDGES, 1)
